# baseline (speedup 1.0000x reference)
import functools
import jax
import jax.numpy as jnp
from jax import lax
from jax.experimental import pallas as pl
from jax.experimental.pallas import tpu as pltpu
from jax.experimental.pallas import tpu_sc as plsc

N = 100000
E = 1600000
NC = 2
NS = 16
NW = NC * NS
LN = 16
NPAD = 100352
EPAD = NW * 50176
EPT = EPAD // NW
RPT = NPAD // NS
ZR = 98
RT = 2048
TGRID = NPAD // RT
ERB = EPT // 128


@functools.lru_cache(maxsize=None)
def _mesh():
    return plsc.VectorSubcoreMesh(core_axis_name="c", subcore_axis_name="s",
                                  num_cores=NC, num_subcores=NS)


_SC_PARAMS = pltpu.CompilerParams(needs_layout_passes=False,
                                  use_tc_tiling_on_sc=False)


def _a1_body(x_ref, oh_ref, temb_ref, w1a_ref, w1b_ref, a1s_ref, a1d_ref,
             h0_ref, h1_ref, h2_ref, h3_ref, pa_ref, ws_ref):
    te = jnp.dot(oh_ref[...], temb_ref[...], preferred_element_type=jnp.float32)
    h = jnp.dot(x_ref[...], w1a_ref[...], preferred_element_type=jnp.float32)
    h = h + jnp.dot(te, w1b_ref[...], preferred_element_type=jnp.float32)
    h0_ref[...] = h[:, 0:16]
    h1_ref[...] = h[:, 16:32]
    h2_ref[...] = h[:, 32:48]
    h3_ref[...] = h[:, 48:64]
    ts = h * a1s_ref[...]
    td = h * a1d_ref[...]
    as0 = jnp.sum(ts[:, :32], axis=1, keepdims=True)
    as1 = jnp.sum(ts[:, 32:], axis=1, keepdims=True)
    ad0 = jnp.sum(td[:, :32], axis=1, keepdims=True)
    ad1 = jnp.sum(td[:, 32:], axis=1, keepdims=True)
    v0 = as0 + ad0
    v1 = as1 + ad1
    w0 = jnp.exp(jnp.maximum(v0, 0.2 * v0))
    w1 = jnp.exp(jnp.maximum(v1, 0.2 * v1))
    pa_ref[...] = jnp.concatenate(
        [as0, as1, ad0, ad1, jnp.zeros((RT, 12), jnp.float32)], axis=1)
    ws_ref[...] = jnp.concatenate([w0, w1], axis=1)


def _a1(xp, oh, temb, w1a, w1b, a1s, a1d):
    full = lambda shp: pl.BlockSpec(shp, lambda i: tuple(0 for _ in shp))
    row = lambda m: pl.BlockSpec((RT, m), lambda i: (i, 0))
    return pl.pallas_call(
        _a1_body,
        grid=(TGRID,),
        in_specs=[row(8), row(8), full((8, 16)), full((8, 64)),
                  full((16, 64)), full((1, 64)), full((1, 64))],
        out_specs=[row(LN)] * 4 + [row(LN), row(2)],
        out_shape=([jax.ShapeDtypeStruct((NPAD, LN), jnp.float32)] * 4
                   + [jax.ShapeDtypeStruct((NPAD, LN), jnp.float32),
                      jax.ShapeDtypeStruct((NPAD, 2), jnp.float32)]),
    )(xp, oh, temb, w1a, w1b, a1s, a1d)


def _make_b1(H):
    KC = 256
    NB = KC // 128
    NCH = EPT // KC
    PW = LN
    out_type = ([jax.ShapeDtypeStruct((EPAD,), jnp.float32) for _ in range(H)]
                + [jax.ShapeDtypeStruct((NC, NPAD, LN), jnp.float32)])
    scratch = ([pltpu.VMEM((NB, 128), jnp.int32) for _ in range(4)]
               + [pltpu.VMEM((KC, PW), jnp.float32) for _ in range(4)]
               + [pltpu.VMEM((KC, LN), jnp.float32)]
               + [pltpu.VMEM((KC,), jnp.float32) for _ in range(H)]
               + [pltpu.VMEM((ZR, LN), jnp.float32),
                  pltpu.VMEM_SHARED((NPAD, LN), jnp.float32)]
               + [pltpu.SemaphoreType.DMA for _ in range(4)])

    def body(ei_hbm, pa_hbm, *rest):
        w_hbm = rest[:H]
        dp_hbm = rest[H]
        r = rest[H + 1:]
        srcv = r[0:2]
        dstv = r[2:4]
        psrc = r[4:6]
        pdst = r[6:8]
        wrows = r[8]
        whs = r[9:9 + H]
        zbuf, den_sp, si0, si1, sg0, sg1 = r[9 + H:]
        si = (si0, si1)
        sg = (sg0, sg1)
        cid = lax.axis_index("c")
        sid = lax.axis_index("s")
        wid = cid * NS + sid
        zero16 = jnp.zeros((LN,), jnp.float32)
        iota16 = lax.iota(jnp.int32, 16)

        def zrow(i, _):
            zbuf[i, :] = zero16
            return 0
        lax.fori_loop(0, ZR, zrow, 0)

        def wz(i, _):
            wrows[i, :] = zero16
            return 0
        lax.fori_loop(0, KC, wz, 0)

        for j in range(RPT // ZR):
            pltpu.sync_copy(zbuf, den_sp.at[pl.ds(sid * RPT + j * ZR, ZR)])
        plsc.subcore_barrier()

        def idx_start(ci, p):
            row0 = wid * ERB + ci * NB
            pltpu.async_copy(ei_hbm.at[0, pl.ds(row0, NB)], srcv[p], si[p])
            pltpu.async_copy(ei_hbm.at[1, pl.ds(row0, NB)], dstv[p], si[p])

        def idx_wait(ci, p):
            row0 = wid * ERB + ci * NB
            pltpu.make_async_copy(ei_hbm.at[0, pl.ds(row0, NB)], srcv[p],
                                  si[p]).wait()
            pltpu.make_async_copy(ei_hbm.at[1, pl.ds(row0, NB)], dstv[p],
                                  si[p]).wait()

        def g_start(p):
            for b in range(NB):
                pltpu.async_copy(pa_hbm.at[srcv[p].at[b]],
                                 psrc[p].at[pl.ds(b * 128, 128)], sg[p])
                pltpu.async_copy(pa_hbm.at[dstv[p].at[b]],
                                 pdst[p].at[pl.ds(b * 128, 128)], sg[p])

        def g_wait(p):
            for b in range(NB):
                pltpu.make_async_copy(pa_hbm.at[srcv[p].at[b]],
                                      psrc[p].at[pl.ds(b * 128, 128)],
                                      sg[p]).wait()
                pltpu.make_async_copy(pa_hbm.at[dstv[p].at[b]],
                                      pdst[p].at[pl.ds(b * 128, 128)],
                                      sg[p]).wait()

        def process(ci, p):
            off = wid * EPT + ci * KC
            for h in range(H):
                colh = jnp.full((16,), h, jnp.int32)
                colad = jnp.full((16,), H + h, jnp.int32)

                def grp(g, _, p=p, h=h, colh=colh, colad=colad):
                    rr = g * 16 + iota16
                    a_s = plsc.load_gather(psrc[p], [rr, colh])
                    a_d = plsc.load_gather(pdst[p], [rr, colad])
                    v = a_s + a_d
                    w = jnp.exp(jnp.maximum(v, 0.2 * v))
                    whs[h][pl.ds(g * 16, 16)] = w
                    plsc.store_scatter(wrows, [rr, colh], w)
                    return 0
                lax.fori_loop(0, KC // 16, grp, 0, unroll=2)
                pltpu.sync_copy(whs[h], w_hbm[h].at[pl.ds(off, KC)])
            for b in range(NB):
                pltpu.sync_copy(wrows.at[pl.ds(b * 128, 128)],
                                den_sp.at[dstv[p].at[b]], add=True)

        idx_start(0, 0)
        idx_wait(0, 0)
        g_start(0)

        def pair(gi, _):
            for p in range(2):
                c = 2 * gi + p
                q = 1 - p

                @pl.when(c + 1 < NCH)
                def _():
                    idx_start(c + 1, q)
                g_wait(p)

                @pl.when(c + 1 < NCH)
                def _():
                    idx_wait(c + 1, q)
                    g_start(q)
                process(c, p)
            return 0
        lax.fori_loop(0, NCH // 2, pair, 0)
        plsc.subcore_barrier()
        pltpu.sync_copy(den_sp.at[pl.ds(sid * RPT, RPT)],
                        dp_hbm.at[cid, pl.ds(sid * RPT, RPT)])

    return pl.kernel(body, out_type=out_type, mesh=_mesh(),
                     scratch_types=scratch, compiler_params=_SC_PARAMS)


def _make_b2(H, head_of):
    KC = 512
    NB = KC // 128
    NCH = EPT // KC
    out_type = [jax.ShapeDtypeStruct((NC, NPAD, LN), jnp.float32)
                for _ in range(4)]
    scratch = ([pltpu.VMEM((NB, 128), jnp.int32) for _ in range(4)]
               + [pltpu.VMEM((KC,), jnp.float32) for _ in range(2)]
               + [pltpu.VMEM((KC, LN), jnp.float32) for _ in range(2)]
               + [pltpu.VMEM((ZR, LN), jnp.float32),
                  pltpu.VMEM_SHARED((NPAD, LN), jnp.float32)]
               + [pltpu.SemaphoreType.DMA for _ in range(4)])

    def body(ei_hbm, *rest):
        hs = rest[:4]
        w_hbm = rest[4:4 + H]
        accs = rest[4 + H:8 + H]
        r = rest[8 + H:]
        srcv = r[0:2]
        dstv = r[2:4]
        wb = r[4:6]
        rows = r[6:8]
        zbuf, acc_sp, si0, si1, sg0, sg1 = r[8:]
        si = (si0, si1)
        sg = (sg0, sg1)
        cid = lax.axis_index("c")
        sid = lax.axis_index("s")
        wid = cid * NS + sid
        zero16 = jnp.zeros((LN,), jnp.float32)

        def zrow(i, _):
            zbuf[i, :] = zero16
            return 0
        lax.fori_loop(0, ZR, zrow, 0)

        for s in range(4):
            table = hs[s]
            whb = w_hbm[head_of[s]]

            def idx_start(ci, p, whb=whb):
                off = wid * EPT + ci * KC
                row0 = wid * ERB + ci * NB
                pltpu.async_copy(ei_hbm.at[0, pl.ds(row0, NB)], srcv[p], si[p])
                pltpu.async_copy(ei_hbm.at[1, pl.ds(row0, NB)], dstv[p], si[p])
                pltpu.async_copy(whb.at[pl.ds(off, KC)], wb[p], si[p])

            def idx_wait(ci, p, whb=whb):
                off = wid * EPT + ci * KC
                row0 = wid * ERB + ci * NB
                pltpu.make_async_copy(ei_hbm.at[0, pl.ds(row0, NB)], srcv[p],
                                      si[p]).wait()
                pltpu.make_async_copy(ei_hbm.at[1, pl.ds(row0, NB)], dstv[p],
                                      si[p]).wait()
                pltpu.make_async_copy(whb.at[pl.ds(off, KC)], wb[p],
                                      si[p]).wait()

            def g_start(p, table=table):
                for b in range(NB):
                    pltpu.async_copy(table.at[srcv[p].at[b]],
                                     rows[p].at[pl.ds(b * 128, 128)], sg[p])

            def g_wait(p, table=table):
                for b in range(NB):
                    pltpu.make_async_copy(table.at[srcv[p].at[b]],
                                          rows[p].at[pl.ds(b * 128, 128)],
                                          sg[p]).wait()

            def process(p):
                def scale(g, _, p=p):
                    base = g * 16
                    w16 = wb[p][pl.ds(base, 16)]
                    for j in range(16):
                        rows[p][base + j, :] = rows[p][base + j, :] * w16[j]
                    return 0
                lax.fori_loop(0, KC // 16, scale, 0, unroll=4)
                for b in range(NB):
                    pltpu.sync_copy(rows[p].at[pl.ds(b * 128, 128)],
                                    acc_sp.at[dstv[p].at[b]], add=True)

            for j in range(RPT // ZR):
                pltpu.sync_copy(zbuf, acc_sp.at[pl.ds(sid * RPT + j * ZR, ZR)])
            plsc.subcore_barrier()

            idx_start(0, 0)
            idx_wait(0, 0)
            g_start(0)

            def pair(gi, _):
                for p in range(2):
                    c = 2 * gi + p
                    q = 1 - p

                    @pl.when(c + 1 < NCH)
                    def _():
                        idx_start(c + 1, q)
                    g_wait(p)

                    @pl.when(c + 1 < NCH)
                    def _():
                        idx_wait(c + 1, q)
                        g_start(q)
                    process(p)
                return 0
            lax.fori_loop(0, NCH // 2, pair, 0)
            plsc.subcore_barrier()
            pltpu.sync_copy(acc_sp.at[pl.ds(sid * RPT, RPT)],
                            accs[s].at[cid, pl.ds(sid * RPT, RPT)])
            plsc.subcore_barrier()

    return pl.kernel(body, out_type=out_type, mesh=_mesh(),
                     scratch_types=scratch, compiler_params=_SC_PARAMS)


def _c1_body(ws_ref, dp1_ref, a0, a1, a2, a3, h0, h1, h2, h3,
             w2_ref, a2s_ref, a2d_ref, b1_ref,
             g0_ref, g1_ref, g2_ref, g3_ref, pa_ref, ws2_ref):
    ws = ws_ref[...]
    dp = dp1_ref[...]
    accs = (a0, a1, a2, a3)
    hss = (h0, h1, h2, h3)
    rden = []
    for h in range(2):
        den = dp[0][:, h:h + 1] + dp[1][:, h:h + 1] + ws[:, h:h + 1]
        rden.append(1.0 / (den + 1e-16))
    parts = []
    for s in range(4):
        h = s // 2
        a = accs[s][...]
        tot = a[0] + a[1] + ws[:, h:h + 1] * hss[s][...]
        col = jax.nn.relu(tot * rden[h] + b1_ref[:, 16 * s:16 * (s + 1)])
        parts.append(jnp.dot(col, w2_ref[16 * s:16 * (s + 1), :],
                             preferred_element_type=jnp.float32))
    h2o = parts[0] + parts[1] + parts[2] + parts[3]
    g0_ref[...] = h2o[:, 0:16]
    g1_ref[...] = h2o[:, 16:32]
    g2_ref[...] = h2o[:, 32:48]
    g3_ref[...] = h2o[:, 48:64]
    as2 = jnp.sum(h2o * a2s_ref[...], axis=1, keepdims=True)
    ad2 = jnp.sum(h2o * a2d_ref[...], axis=1, keepdims=True)
    v = as2 + ad2
    w2s = jnp.exp(jnp.maximum(v, 0.2 * v))
    pa_ref[...] = jnp.concatenate(
        [as2, ad2, jnp.zeros((RT, 14), jnp.float32)], axis=1)
    ws2_ref[...] = jnp.concatenate([w2s, w2s], axis=1)


def _c1(ws1, dp1, a1accs, h1s, W2, a2s, a2d, b1r):
    full = lambda shp: pl.BlockSpec(shp, lambda i: tuple(0 for _ in shp))
    row = lambda m: pl.BlockSpec((RT, m), lambda i: (i, 0))
    dp_spec = pl.BlockSpec((NC, RT, LN), lambda i: (0, i, 0))
    return pl.pallas_call(
        _c1_body,
        grid=(TGRID,),
        in_specs=([row(2), dp_spec] + [dp_spec] * 4 + [row(LN)] * 4
                  + [full((64, 64)), full((1, 64)), full((1, 64)),
                     full((1, 64))]),
        out_specs=[row(LN)] * 4 + [row(LN), row(2)],
        out_shape=([jax.ShapeDtypeStruct((NPAD, LN), jnp.float32)] * 4
                   + [jax.ShapeDtypeStruct((NPAD, LN), jnp.float32),
                      jax.ShapeDtypeStruct((NPAD, 2), jnp.float32)]),
    )(ws1, dp1, *a1accs, *h1s, W2, a2s, a2d, b1r)


def _c2_body(ws_ref, dp2_ref, a0, a1, a2, a3, h0, h1, h2, h3, b2_ref,
             lnw_ref, lnb_ref, o_ref):
    ws = ws_ref[...]
    dp = dp2_ref[...]
    den = dp[0][:, 0:1] + dp[1][:, 0:1] + ws[:, 0:1]
    rden = 1.0 / (den + 1e-16)
    accs = (a0, a1, a2, a3)
    hss = (h0, h1, h2, h3)
    cols = []
    for s in range(4):
        a = accs[s][...]
        tot = a[0] + a[1] + ws[:, 0:1] * hss[s][...]
        cols.append(tot * rden + b2_ref[:, 16 * s:16 * (s + 1)])
    out = jnp.concatenate(cols, axis=1)
    mu = jnp.mean(out, axis=-1, keepdims=True)
    var = jnp.mean((out - mu) ** 2, axis=-1, keepdims=True)
    o_ref[...] = (out - mu) * lax.rsqrt(var + 1e-5) * lnw_ref[...] + lnb_ref[...]


def _c2(ws2, dp2, a2accs, h2s, b2r, lnwr, lnbr):
    full = lambda shp: pl.BlockSpec(shp, lambda i: tuple(0 for _ in shp))
    row = lambda m: pl.BlockSpec((RT, m), lambda i: (i, 0))
    dp_spec = pl.BlockSpec((NC, RT, LN), lambda i: (0, i, 0))
    return pl.pallas_call(
        _c2_body,
        grid=(TGRID,),
        in_specs=([row(2), dp_spec] + [dp_spec] * 4 + [row(LN)] * 4
                  + [full((1, 64)), full((1, 64)), full((1, 64))]),
        out_specs=row(64),
        out_shape=jax.ShapeDtypeStruct((NPAD, 64), jnp.float32),
    )(ws2, dp2, *a2accs, *h2s, b2r, lnwr, lnbr)


_make_b1 = functools.lru_cache(maxsize=None)(_make_b1)
_make_b2 = functools.lru_cache(maxsize=None)(_make_b2)


def kernel(x, edge_index, type_ids, type_emb, W1, a_src1, a_dst1, b1,
           W2, a_src2, a_dst2, b2, ln_w, ln_b):
    f32 = jnp.float32
    xp = jnp.zeros((NPAD, 8), f32).at[:N, :5].set(x)
    tid = jnp.zeros((NPAD,), jnp.int32).at[:N].set(type_ids)
    oh = (tid[:, None] == jnp.arange(8, dtype=jnp.int32)[None, :]).astype(f32)
    w1a = jnp.zeros((8, 64), f32).at[:5].set(W1[:5])
    w1b = W1[5:]
    a1s = a_src1.reshape(1, 64)
    a1d = a_dst1.reshape(1, 64)
    a2s = a_src2.reshape(1, 64)
    a2d = a_dst2.reshape(1, 64)
    b1r = b1.reshape(1, 64)
    b2r = b2.reshape(1, 64)
    lnwr = ln_w.reshape(1, 64)
    lnbr = ln_b.reshape(1, 64)

    pad_idx = jnp.full((2, EPAD - E), N, jnp.int32)
    ei2d = jnp.concatenate([edge_index, pad_idx], axis=1).reshape(
        2, EPAD // 128, 128)

    h1s0, h1s1, h1s2, h1s3, pa1, ws1 = _a1(xp, oh, type_emb, w1a, w1b,
                                           a1s, a1d)
    h1s = [h1s0, h1s1, h1s2, h1s3]
    w10, w11, dp1 = _make_b1(2)(ei2d, pa1)
    a1accs = _make_b2(2, (0, 0, 1, 1))(ei2d, *h1s, w10, w11)
    h2s0, h2s1, h2s2, h2s3, pa2, ws2 = _c1(ws1, dp1, a1accs, h1s,
                                           W2, a2s, a2d, b1r)
    h2s = [h2s0, h2s1, h2s2, h2s3]
    w20, dp2 = _make_b1(1)(ei2d, pa2)
    a2accs = _make_b2(1, (0, 0, 0, 0))(ei2d, *h2s, w20)
    out = _c2(ws2, dp2, a2accs, h2s, b2r, lnwr, lnbr)
    return out[:N]


# speedup vs baseline: 1.2865x; 1.2865x over previous
import functools
import jax
import jax.numpy as jnp
from jax import lax
from jax.experimental import pallas as pl
from jax.experimental.pallas import tpu as pltpu
from jax.experimental.pallas import tpu_sc as plsc

N = 100000
E = 1600000
NC = 2
NS = 16
NW = NC * NS
LN = 16
NPAD = 100352
EPAD = NW * 50176
EPT = EPAD // NW
RPT = NPAD // NS
ZR = 98
RT = 2048
TGRID = NPAD // RT
ERB = EPT // 128


@functools.lru_cache(maxsize=None)
def _mesh():
    return plsc.VectorSubcoreMesh(core_axis_name="c", subcore_axis_name="s",
                                  num_cores=NC, num_subcores=NS)


_SC_PARAMS = pltpu.CompilerParams(needs_layout_passes=False,
                                  use_tc_tiling_on_sc=False)


def _a1_body(x_ref, oh_ref, temb_ref, w1a_ref, w1b_ref, a1s_ref, a1d_ref,
             h0_ref, h1_ref, h2_ref, h3_ref, pa_ref, ws_ref):
    te = jnp.dot(oh_ref[...], temb_ref[...], preferred_element_type=jnp.float32)
    h = jnp.dot(x_ref[...], w1a_ref[...], preferred_element_type=jnp.float32)
    h = h + jnp.dot(te, w1b_ref[...], preferred_element_type=jnp.float32)
    h0_ref[...] = h[:, 0:16]
    h1_ref[...] = h[:, 16:32]
    h2_ref[...] = h[:, 32:48]
    h3_ref[...] = h[:, 48:64]
    ts = h * a1s_ref[...]
    td = h * a1d_ref[...]
    as0 = jnp.sum(ts[:, :32], axis=1, keepdims=True)
    as1 = jnp.sum(ts[:, 32:], axis=1, keepdims=True)
    ad0 = jnp.sum(td[:, :32], axis=1, keepdims=True)
    ad1 = jnp.sum(td[:, 32:], axis=1, keepdims=True)
    v0 = as0 + ad0
    v1 = as1 + ad1
    w0 = jnp.exp(jnp.maximum(v0, 0.2 * v0))
    w1 = jnp.exp(jnp.maximum(v1, 0.2 * v1))
    pa_ref[...] = jnp.concatenate(
        [as0, as1, ad0, ad1, jnp.zeros((RT, 12), jnp.float32)], axis=1)
    ws_ref[...] = jnp.concatenate([w0, w1], axis=1)


def _a1(xp, oh, temb, w1a, w1b, a1s, a1d):
    full = lambda shp: pl.BlockSpec(shp, lambda i: tuple(0 for _ in shp))
    row = lambda m: pl.BlockSpec((RT, m), lambda i: (i, 0))
    return pl.pallas_call(
        _a1_body,
        grid=(TGRID,),
        in_specs=[row(8), row(8), full((8, 16)), full((8, 64)),
                  full((16, 64)), full((1, 64)), full((1, 64))],
        out_specs=[row(LN)] * 4 + [row(LN), row(2)],
        out_shape=([jax.ShapeDtypeStruct((NPAD, LN), jnp.float32)] * 4
                   + [jax.ShapeDtypeStruct((NPAD, LN), jnp.float32),
                      jax.ShapeDtypeStruct((NPAD, 2), jnp.float32)]),
    )(xp, oh, temb, w1a, w1b, a1s, a1d)


def _make_b1(H):
    KC = 256
    NB = KC // 128
    NCH = EPT // KC
    PW = LN
    out_type = ([jax.ShapeDtypeStruct((EPAD,), jnp.float32) for _ in range(H)]
                + [jax.ShapeDtypeStruct((NC, NPAD, LN), jnp.float32)])
    scratch = ([pltpu.VMEM((NB, 128), jnp.int32) for _ in range(4)]
               + [pltpu.VMEM((KC, PW), jnp.float32) for _ in range(4)]
               + [pltpu.VMEM((KC, LN), jnp.float32)]
               + [pltpu.VMEM((KC,), jnp.float32) for _ in range(H)]
               + [pltpu.VMEM((ZR, LN), jnp.float32),
                  pltpu.VMEM_SHARED((NPAD, LN), jnp.float32)]
               + [pltpu.SemaphoreType.DMA for _ in range(4)])

    def body(ei_hbm, pa_hbm, *rest):
        w_hbm = rest[:H]
        dp_hbm = rest[H]
        r = rest[H + 1:]
        srcv = r[0:2]
        dstv = r[2:4]
        psrc = r[4:6]
        pdst = r[6:8]
        wrows = r[8]
        whs = r[9:9 + H]
        zbuf, den_sp, si0, si1, sg0, sg1 = r[9 + H:]
        si = (si0, si1)
        sg = (sg0, sg1)
        cid = lax.axis_index("c")
        sid = lax.axis_index("s")
        wid = cid * NS + sid
        zero16 = jnp.zeros((LN,), jnp.float32)
        iota16 = lax.iota(jnp.int32, 16)

        def zrow(i, _):
            zbuf[i, :] = zero16
            return 0
        lax.fori_loop(0, ZR, zrow, 0)

        def wz(i, _):
            wrows[i, :] = zero16
            return 0
        lax.fori_loop(0, KC, wz, 0)

        for j in range(RPT // ZR):
            pltpu.sync_copy(zbuf, den_sp.at[pl.ds(sid * RPT + j * ZR, ZR)])
        plsc.subcore_barrier()

        def idx_start(ci, p):
            row0 = wid * ERB + ci * NB
            pltpu.async_copy(ei_hbm.at[0, pl.ds(row0, NB)], srcv[p], si[p])
            pltpu.async_copy(ei_hbm.at[1, pl.ds(row0, NB)], dstv[p], si[p])

        def idx_wait(ci, p):
            row0 = wid * ERB + ci * NB
            pltpu.make_async_copy(ei_hbm.at[0, pl.ds(row0, NB)], srcv[p],
                                  si[p]).wait()
            pltpu.make_async_copy(ei_hbm.at[1, pl.ds(row0, NB)], dstv[p],
                                  si[p]).wait()

        def g_start(p):
            for b in range(NB):
                pltpu.async_copy(pa_hbm.at[srcv[p].at[b]],
                                 psrc[p].at[pl.ds(b * 128, 128)], sg[p])
                pltpu.async_copy(pa_hbm.at[dstv[p].at[b]],
                                 pdst[p].at[pl.ds(b * 128, 128)], sg[p])

        def g_wait(p):
            for b in range(NB):
                pltpu.make_async_copy(pa_hbm.at[srcv[p].at[b]],
                                      psrc[p].at[pl.ds(b * 128, 128)],
                                      sg[p]).wait()
                pltpu.make_async_copy(pa_hbm.at[dstv[p].at[b]],
                                      pdst[p].at[pl.ds(b * 128, 128)],
                                      sg[p]).wait()

        def process(ci, p):
            off = wid * EPT + ci * KC
            for h in range(H):
                colh = jnp.full((16,), h, jnp.int32)
                colad = jnp.full((16,), H + h, jnp.int32)

                def grp(g, _, p=p, h=h, colh=colh, colad=colad):
                    rr = g * 16 + iota16
                    a_s = plsc.load_gather(psrc[p], [rr, colh])
                    a_d = plsc.load_gather(pdst[p], [rr, colad])
                    v = a_s + a_d
                    w = jnp.exp(jnp.maximum(v, 0.2 * v))
                    whs[h][pl.ds(g * 16, 16)] = w
                    plsc.store_scatter(wrows, [rr, colh], w)
                    return 0
                lax.fori_loop(0, KC // 16, grp, 0, unroll=2)
                pltpu.sync_copy(whs[h], w_hbm[h].at[pl.ds(off, KC)])
            for b in range(NB):
                pltpu.sync_copy(wrows.at[pl.ds(b * 128, 128)],
                                den_sp.at[dstv[p].at[b]], add=True)

        idx_start(0, 0)
        idx_wait(0, 0)
        g_start(0)

        def pair(gi, _):
            for p in range(2):
                c = 2 * gi + p
                q = 1 - p

                @pl.when(c + 1 < NCH)
                def _():
                    idx_start(c + 1, q)
                g_wait(p)

                @pl.when(c + 1 < NCH)
                def _():
                    idx_wait(c + 1, q)
                    g_start(q)
                process(c, p)
            return 0
        lax.fori_loop(0, NCH // 2, pair, 0)
        plsc.subcore_barrier()
        pltpu.sync_copy(den_sp.at[pl.ds(sid * RPT, RPT)],
                        dp_hbm.at[cid, pl.ds(sid * RPT, RPT)])

    return pl.kernel(body, out_type=out_type, mesh=_mesh(),
                     scratch_types=scratch, compiler_params=_SC_PARAMS)


def _make_b2(H, head_of):
    KC = 512
    NB = KC // 128
    NCH = EPT // KC
    out_type = [jax.ShapeDtypeStruct((NC, NPAD, LN), jnp.float32)
                for _ in range(4)]
    scratch = ([pltpu.VMEM((NB, 128), jnp.int32) for _ in range(4)]
               + [pltpu.VMEM((KC,), jnp.float32) for _ in range(2)]
               + [pltpu.VMEM((KC, LN), jnp.float32) for _ in range(2)]
               + [pltpu.VMEM((ZR, LN), jnp.float32),
                  pltpu.VMEM_SHARED((NPAD, LN), jnp.float32)]
               + [pltpu.SemaphoreType.DMA for _ in range(4)])

    def body(ei_hbm, *rest):
        hs = rest[:4]
        w_hbm = rest[4:4 + H]
        accs = rest[4 + H:8 + H]
        r = rest[8 + H:]
        srcv = r[0:2]
        dstv = r[2:4]
        wb = r[4:6]
        rows = r[6:8]
        zbuf, acc_sp, si0, si1, sg0, sg1 = r[8:]
        si = (si0, si1)
        sg = (sg0, sg1)
        cid = lax.axis_index("c")
        sid = lax.axis_index("s")
        wid = cid * NS + sid
        zero16 = jnp.zeros((LN,), jnp.float32)

        def zrow(i, _):
            zbuf[i, :] = zero16
            return 0
        lax.fori_loop(0, ZR, zrow, 0)

        for s in range(4):
            table = hs[s]
            whb = w_hbm[head_of[s]]

            def idx_start(ci, p, whb=whb):
                off = wid * EPT + ci * KC
                row0 = wid * ERB + ci * NB
                pltpu.async_copy(ei_hbm.at[0, pl.ds(row0, NB)], srcv[p], si[p])
                pltpu.async_copy(ei_hbm.at[1, pl.ds(row0, NB)], dstv[p], si[p])
                pltpu.async_copy(whb.at[pl.ds(off, KC)], wb[p], si[p])

            def idx_wait(ci, p, whb=whb):
                off = wid * EPT + ci * KC
                row0 = wid * ERB + ci * NB
                pltpu.make_async_copy(ei_hbm.at[0, pl.ds(row0, NB)], srcv[p],
                                      si[p]).wait()
                pltpu.make_async_copy(ei_hbm.at[1, pl.ds(row0, NB)], dstv[p],
                                      si[p]).wait()
                pltpu.make_async_copy(whb.at[pl.ds(off, KC)], wb[p],
                                      si[p]).wait()

            def g_start(p, table=table):
                for b in range(NB):
                    pltpu.async_copy(table.at[srcv[p].at[b]],
                                     rows[p].at[pl.ds(b * 128, 128)], sg[p])

            def g_wait(p, table=table):
                for b in range(NB):
                    pltpu.make_async_copy(table.at[srcv[p].at[b]],
                                          rows[p].at[pl.ds(b * 128, 128)],
                                          sg[p]).wait()

            def process(p):
                def scale(g, _, p=p):
                    base = g * 16
                    w16 = wb[p][pl.ds(base, 16)]
                    for j in range(16):
                        rows[p][base + j, :] = rows[p][base + j, :] * w16[j]
                    return 0
                lax.fori_loop(0, KC // 16, scale, 0, unroll=2)
                for b in range(NB):
                    pltpu.sync_copy(rows[p].at[pl.ds(b * 128, 128)],
                                    acc_sp.at[dstv[p].at[b]], add=True)

            for j in range(RPT // ZR):
                pltpu.sync_copy(zbuf, acc_sp.at[pl.ds(sid * RPT + j * ZR, ZR)])
            plsc.subcore_barrier()

            idx_start(0, 0)
            idx_wait(0, 0)
            g_start(0)

            def pair(gi, _):
                for p in range(2):
                    c = 2 * gi + p
                    q = 1 - p

                    @pl.when(c + 1 < NCH)
                    def _():
                        idx_start(c + 1, q)
                    g_wait(p)

                    @pl.when(c + 1 < NCH)
                    def _():
                        idx_wait(c + 1, q)
                        g_start(q)
                    process(p)
                return 0
            lax.fori_loop(0, NCH // 2, pair, 0)
            plsc.subcore_barrier()
            pltpu.sync_copy(acc_sp.at[pl.ds(sid * RPT, RPT)],
                            accs[s].at[cid, pl.ds(sid * RPT, RPT)])
            plsc.subcore_barrier()

    return pl.kernel(body, out_type=out_type, mesh=_mesh(),
                     scratch_types=scratch, compiler_params=_SC_PARAMS)


def _c1_body(ws_ref, dp1_ref, a0, a1, a2, a3, h0, h1, h2, h3,
             w2_ref, a2s_ref, a2d_ref, b1_ref,
             g0_ref, g1_ref, g2_ref, g3_ref, pa_ref, ws2_ref):
    ws = ws_ref[...]
    dp = dp1_ref[...]
    accs = (a0, a1, a2, a3)
    hss = (h0, h1, h2, h3)
    rden = []
    for h in range(2):
        den = dp[0][:, h:h + 1] + dp[1][:, h:h + 1] + ws[:, h:h + 1]
        rden.append(1.0 / (den + 1e-16))
    parts = []
    for s in range(4):
        h = s // 2
        a = accs[s][...]
        tot = a[0] + a[1] + ws[:, h:h + 1] * hss[s][...]
        col = jax.nn.relu(tot * rden[h] + b1_ref[:, 16 * s:16 * (s + 1)])
        parts.append(jnp.dot(col, w2_ref[16 * s:16 * (s + 1), :],
                             preferred_element_type=jnp.float32))
    h2o = parts[0] + parts[1] + parts[2] + parts[3]
    g0_ref[...] = h2o[:, 0:16]
    g1_ref[...] = h2o[:, 16:32]
    g2_ref[...] = h2o[:, 32:48]
    g3_ref[...] = h2o[:, 48:64]
    as2 = jnp.sum(h2o * a2s_ref[...], axis=1, keepdims=True)
    ad2 = jnp.sum(h2o * a2d_ref[...], axis=1, keepdims=True)
    v = as2 + ad2
    w2s = jnp.exp(jnp.maximum(v, 0.2 * v))
    pa_ref[...] = jnp.concatenate(
        [as2, ad2, jnp.zeros((RT, 14), jnp.float32)], axis=1)
    ws2_ref[...] = jnp.concatenate([w2s, w2s], axis=1)


def _c1(ws1, dp1, a1accs, h1s, W2, a2s, a2d, b1r):
    full = lambda shp: pl.BlockSpec(shp, lambda i: tuple(0 for _ in shp))
    row = lambda m: pl.BlockSpec((RT, m), lambda i: (i, 0))
    dp_spec = pl.BlockSpec((NC, RT, LN), lambda i: (0, i, 0))
    return pl.pallas_call(
        _c1_body,
        grid=(TGRID,),
        in_specs=([row(2), dp_spec] + [dp_spec] * 4 + [row(LN)] * 4
                  + [full((64, 64)), full((1, 64)), full((1, 64)),
                     full((1, 64))]),
        out_specs=[row(LN)] * 4 + [row(LN), row(2)],
        out_shape=([jax.ShapeDtypeStruct((NPAD, LN), jnp.float32)] * 4
                   + [jax.ShapeDtypeStruct((NPAD, LN), jnp.float32),
                      jax.ShapeDtypeStruct((NPAD, 2), jnp.float32)]),
    )(ws1, dp1, *a1accs, *h1s, W2, a2s, a2d, b1r)


def _c2_body(ws_ref, dp2_ref, a0, a1, a2, a3, h0, h1, h2, h3, b2_ref,
             lnw_ref, lnb_ref, o_ref):
    ws = ws_ref[...]
    dp = dp2_ref[...]
    den = dp[0][:, 0:1] + dp[1][:, 0:1] + ws[:, 0:1]
    rden = 1.0 / (den + 1e-16)
    accs = (a0, a1, a2, a3)
    hss = (h0, h1, h2, h3)
    cols = []
    for s in range(4):
        a = accs[s][...]
        tot = a[0] + a[1] + ws[:, 0:1] * hss[s][...]
        cols.append(tot * rden + b2_ref[:, 16 * s:16 * (s + 1)])
    out = jnp.concatenate(cols, axis=1)
    mu = jnp.mean(out, axis=-1, keepdims=True)
    var = jnp.mean((out - mu) ** 2, axis=-1, keepdims=True)
    o_ref[...] = (out - mu) * lax.rsqrt(var + 1e-5) * lnw_ref[...] + lnb_ref[...]


def _c2(ws2, dp2, a2accs, h2s, b2r, lnwr, lnbr):
    full = lambda shp: pl.BlockSpec(shp, lambda i: tuple(0 for _ in shp))
    row = lambda m: pl.BlockSpec((RT, m), lambda i: (i, 0))
    dp_spec = pl.BlockSpec((NC, RT, LN), lambda i: (0, i, 0))
    return pl.pallas_call(
        _c2_body,
        grid=(TGRID,),
        in_specs=([row(2), dp_spec] + [dp_spec] * 4 + [row(LN)] * 4
                  + [full((1, 64)), full((1, 64)), full((1, 64))]),
        out_specs=row(64),
        out_shape=jax.ShapeDtypeStruct((NPAD, 64), jnp.float32),
    )(ws2, dp2, *a2accs, *h2s, b2r, lnwr, lnbr)


_make_b1 = functools.lru_cache(maxsize=None)(_make_b1)
_make_b2 = functools.lru_cache(maxsize=None)(_make_b2)


def kernel(x, edge_index, type_ids, type_emb, W1, a_src1, a_dst1, b1,
           W2, a_src2, a_dst2, b2, ln_w, ln_b):
    f32 = jnp.float32
    xp = jnp.zeros((NPAD, 8), f32).at[:N, :5].set(x)
    tid = jnp.zeros((NPAD,), jnp.int32).at[:N].set(type_ids)
    oh = (tid[:, None] == jnp.arange(8, dtype=jnp.int32)[None, :]).astype(f32)
    w1a = jnp.zeros((8, 64), f32).at[:5].set(W1[:5])
    w1b = W1[5:]
    a1s = a_src1.reshape(1, 64)
    a1d = a_dst1.reshape(1, 64)
    a2s = a_src2.reshape(1, 64)
    a2d = a_dst2.reshape(1, 64)
    b1r = b1.reshape(1, 64)
    b2r = b2.reshape(1, 64)
    lnwr = ln_w.reshape(1, 64)
    lnbr = ln_b.reshape(1, 64)

    pad_idx = jnp.full((2, EPAD - E), N, jnp.int32)
    ei2d = jnp.concatenate([edge_index, pad_idx], axis=1).reshape(
        2, EPAD // 128, 128)

    h1s0, h1s1, h1s2, h1s3, pa1, ws1 = _a1(xp, oh, type_emb, w1a, w1b,
                                           a1s, a1d)
    h1s = [h1s0, h1s1, h1s2, h1s3]
    w10, w11, dp1 = _make_b1(2)(ei2d, pa1)
    a1accs = _make_b2(2, (0, 0, 1, 1))(ei2d, *h1s, w10, w11)
    h2s0, h2s1, h2s2, h2s3, pa2, ws2 = _c1(ws1, dp1, a1accs, h1s,
                                           W2, a2s, a2d, b1r)
    h2s = [h2s0, h2s1, h2s2, h2s3]
    w20, dp2 = _make_b1(1)(ei2d, pa2)
    a2accs = _make_b2(1, (0, 0, 0, 0))(ei2d, *h2s, w20)
    out = _c2(ws2, dp2, a2accs, h2s, b2r, lnwr, lnbr)
    return out[:N]
